# baseline (speedup 1.0000x reference)
import dataclasses
import functools

import jax
import jax.numpy as jnp
from jax import lax
from jax.experimental import pallas as pl
from jax.experimental.pallas import tpu as pltpu
from jax.experimental.pallas import tpu_sc as plsc

_N = 10000
_E = 320000
_D = 128
_NC = 2
_NS = 16
_NW = _NC * _NS
_N_PAD = 10240
_E_PAD = 327680
_CHUNK = 128
_NCHUNKS = _E_PAD // _CHUNK
_CPT0 = 116
_CPT1 = (_NCHUNKS - _NS * _CPT0) // _NS
_RPT = _N_PAD // _NS
_ARA = 624
_ARB = _N - 15 * _ARA

_BLK = 2000
_NBLK = _N // _BLK


def _sc_mesh():
    return plsc.VectorSubcoreMesh(core_axis_name="c", subcore_axis_name="s")


def _sc_compiler_params():
    cp = pltpu.CompilerParams()
    if "needs_layout_passes" in pltpu.CompilerParams.__dataclass_fields__:
        cp = dataclasses.replace(cp, needs_layout_passes=False)
    return cp


def _deg_partials(dst_r, ew_r):
    grp = 16
    cpt = _NCHUNKS // _NW

    @functools.partial(
        pl.kernel,
        mesh=_sc_mesh(),
        out_type=jax.ShapeDtypeStruct((_NC, _N_PAD), jnp.float32),
        scratch_types=[
            pltpu.VMEM_SHARED((_N_PAD,), jnp.float32),
            pltpu.VMEM((cpt, _CHUNK), jnp.int32),
            pltpu.VMEM((cpt, _CHUNK), jnp.float32),
            pltpu.VMEM((_RPT,), jnp.float32),
            pltpu.SemaphoreType.DMA,
        ],
    )
    def k(dst_hbm, ew_hbm, out_hbm, acc, idx_all, ew_all, zbuf, sem):
        c = lax.axis_index("c")
        s = lax.axis_index("s")
        wid = c * _NS + s

        pltpu.sync_copy(dst_hbm.at[pl.ds(wid * cpt, cpt)], idx_all)
        pltpu.sync_copy(ew_hbm.at[pl.ds(wid * cpt, cpt)], ew_all)

        @pl.loop(0, _RPT // 16)
        def _(i):
            zbuf[pl.ds(i * 16, 16)] = jnp.zeros((16,), jnp.float32)

        pltpu.sync_copy(zbuf, acc.at[pl.ds(s * _RPT, _RPT)])
        plsc.subcore_barrier()

        @pl.loop(0, cpt // grp)
        def _(gi):
            for j in range(grp):
                pltpu.async_copy(ew_all.at[gi * grp + j],
                                 acc.at[idx_all.at[gi * grp + j]], sem,
                                 add=True)
            for j in range(grp):
                pltpu.make_async_copy(ew_all.at[gi * grp + j],
                                      acc.at[idx_all.at[gi * grp + j]],
                                      sem).wait()

        plsc.subcore_barrier()
        pltpu.sync_copy(acc.at[pl.ds(s * _RPT, _RPT)],
                        out_hbm.at[c, pl.ds(s * _RPT, _RPT)])

    return k(dst_r, ew_r)


def _scatter_partials(hp, src_r, dst_r, ew_r):
    @functools.partial(
        pl.kernel,
        mesh=_sc_mesh(),
        out_type=jax.ShapeDtypeStruct((_NC, _N, _D), jnp.float32),
        compiler_params=_sc_compiler_params(),
        scratch_types=[
            pltpu.VMEM_SHARED((_N, _D), jnp.float32),
            pltpu.VMEM((_CHUNK,), jnp.int32),
            pltpu.VMEM((_CHUNK,), jnp.int32),
            pltpu.VMEM((_CHUNK,), jnp.int32),
            pltpu.VMEM((_CHUNK,), jnp.int32),
            pltpu.VMEM((_CHUNK,), jnp.int32),
            pltpu.VMEM((_CHUNK,), jnp.int32),
            pltpu.VMEM((_CHUNK,), jnp.float32),
            pltpu.VMEM((_CHUNK,), jnp.float32),
            pltpu.VMEM((_CHUNK,), jnp.float32),
            pltpu.VMEM((_CHUNK, _D), jnp.float32),
            pltpu.VMEM((_CHUNK, _D), jnp.float32),
            pltpu.VMEM((_CHUNK, _D), jnp.float32),
            pltpu.SemaphoreType.DMA,
            pltpu.SemaphoreType.DMA,
            pltpu.SemaphoreType.DMA,
            pltpu.SemaphoreType.DMA,
            pltpu.SemaphoreType.DMA,
            pltpu.SemaphoreType.DMA,
        ],
    )
    def k(hp_hbm, src_hbm, dst_hbm, ew_hbm, out_hbm,
          acc, sv0, sv1, sv2, dv0, dv1, dv2, wv0, wv1, wv2,
          b0, b1, b2, gs0, gs1, gs2, ss0, ss1, ss2):
        c = lax.axis_index("c")
        s = lax.axis_index("s")
        bufs = (b0, b1, b2)
        srcv = (sv0, sv1, sv2)
        dstv = (dv0, dv1, dv2)
        eww = (wv0, wv1, wv2)
        gs = (gs0, gs1, gs2)
        ss = (ss0, ss1, ss2)

        @pl.loop(0, _CHUNK)
        def _(i):
            for f in range(_D // 16):
                b0[i, pl.ds(f * 16, 16)] = jnp.zeros((16,), jnp.float32)

        @pl.when(s < 15)
        def _():
            @pl.loop(0, _ARA // 104)
            def _(kk):
                pltpu.sync_copy(b0.at[pl.ds(0, 104)],
                                acc.at[pl.ds(s * _ARA + kk * 104, 104)])

        @pl.when(s == 15)
        def _():
            @pl.loop(0, _ARB // _CHUNK)
            def _(kk):
                pltpu.sync_copy(
                    b0, acc.at[pl.ds(15 * _ARA + kk * _CHUNK, _CHUNK)])

        plsc.subcore_barrier()

        def fetch(ch, j):
            pltpu.async_copy(src_hbm.at[ch], srcv[j], gs[j])
            pltpu.async_copy(dst_hbm.at[ch], dstv[j], gs[j])
            pltpu.async_copy(ew_hbm.at[ch], eww[j], gs[j])

        def wait_fetch(ch, j):
            pltpu.make_async_copy(src_hbm.at[ch], srcv[j], gs[j]).wait()
            pltpu.make_async_copy(dst_hbm.at[ch], dstv[j], gs[j]).wait()
            pltpu.make_async_copy(ew_hbm.at[ch], eww[j], gs[j]).wait()

        def gather_rows(j):
            pltpu.async_copy(hp_hbm.at[srcv[j]], bufs[j], gs[j])

        def wait_rows(j):
            pltpu.make_async_copy(hp_hbm.at[srcv[j]], bufs[j], gs[j]).wait()

        def scale(j):
            buf = bufs[j]

            @pl.loop(0, _CHUNK)
            def _(e):
                w16 = plsc.load_gather(eww[j],
                                       [jnp.full((16,), e, jnp.int32)])
                for f in range(_D // 16):
                    sl = pl.ds(f * 16, 16)
                    buf[e, sl] = buf[e, sl] * w16

        def scatter(j):
            pltpu.async_copy(bufs[j], acc.at[dstv[j]], ss[j], add=True)

        def wait_scatter(j):
            pltpu.make_async_copy(bufs[j], acc.at[dstv[j]], ss[j]).wait()

        def pipeline(base, cpt):
            fetch(base, 0)
            wait_fetch(base, 0)
            gather_rows(0)
            fetch(base + 1, 1)
            wait_fetch(base + 1, 1)
            gather_rows(1)
            wait_rows(0)
            scale(0)
            fetch(base + 2, 2)
            scatter(0)
            wait_fetch(base + 2, 2)
            gather_rows(2)
            wait_rows(1)
            scale(1)
            wait_scatter(0)
            fetch(base + 3, 0)
            scatter(1)

            @pl.loop(0, (cpt - 2) // 3)
            def _(i):
                for u in range(3):
                    g = 2 + 3 * i + u
                    j = (2 + u) % 3
                    j1 = (j + 1) % 3
                    j2 = (j + 2) % 3
                    nxt = base + jnp.minimum(g + 1, cpt - 1)
                    nx2 = base + jnp.minimum(g + 2, cpt - 1)
                    wait_fetch(nxt, j1)
                    gather_rows(j1)
                    wait_rows(j)
                    scale(j)
                    wait_scatter(j2)
                    fetch(nx2, j2)
                    scatter(j)

            wait_fetch(base + cpt - 1, (cpt + 1) % 3)
            wait_rows(cpt % 3)
            wait_scatter((cpt - 1) % 3)

        @pl.when(c == 0)
        def _():
            pipeline(s * _CPT0, _CPT0)

        @pl.when(c == 1)
        def _():
            pipeline(_NS * _CPT0 + s * _CPT1, _CPT1)

        plsc.subcore_barrier()

        @pl.when(s < 15)
        def _():
            @pl.loop(0, _ARA // 104)
            def _(kk):
                r0 = s * _ARA + kk * 104
                pltpu.sync_copy(acc.at[pl.ds(r0, 104)],
                                out_hbm.at[c, pl.ds(r0, 104)])

        @pl.when(s == 15)
        def _():
            @pl.loop(0, _ARB // _CHUNK)
            def _(kk):
                r0 = 15 * _ARA + kk * _CHUNK
                pltpu.sync_copy(acc.at[pl.ds(r0, _CHUNK)],
                                out_hbm.at[c, pl.ds(r0, _CHUNK)])

    return k(hp, src_r, dst_r, ew_r)


def _dis_from_deg(deg_parts):
    deg_r = deg_parts.reshape(_NC, _N_PAD // 128, 128)

    def body(deg_ref, out_ref):
        out_ref[...] = lax.rsqrt(deg_ref[0] + deg_ref[1] + 1.0)

    return pl.pallas_call(
        body,
        out_shape=jax.ShapeDtypeStruct((_N_PAD // 128, 128), jnp.float32),
    )(deg_r)


def _mm_scale(x, W, dis):
    def body(x_ref, w_ref, dis_ref, o_ref):
        h = jnp.dot(x_ref[...], w_ref[...], preferred_element_type=jnp.float32)
        o_ref[...] = dis_ref[...] * h

    return pl.pallas_call(
        body,
        grid=(_NBLK,),
        in_specs=[
            pl.BlockSpec((_BLK, _D), lambda i: (i, 0)),
            pl.BlockSpec((_D, _D), lambda i: (0, 0)),
            pl.BlockSpec((_BLK, 1), lambda i: (i, 0)),
        ],
        out_specs=pl.BlockSpec((_BLK, _D), lambda i: (i, 0)),
        out_shape=jax.ShapeDtypeStruct((_N, _D), jnp.float32),
    )(x, W, dis)


def _layer2_mm(s_parts, hp, dis, W2, b1):
    def body(s_ref, hp_ref, dis_ref, w_ref, b_ref, o_ref):
        g = dis_ref[...] * (s_ref[0] + s_ref[1] + hp_ref[...]) + b_ref[...]
        g = jnp.maximum(g, 0.0)
        h2 = jnp.dot(g, w_ref[...], preferred_element_type=jnp.float32)
        o_ref[...] = dis_ref[...] * h2

    return pl.pallas_call(
        body,
        grid=(_NBLK,),
        in_specs=[
            pl.BlockSpec((_NC, _BLK, _D), lambda i: (0, i, 0)),
            pl.BlockSpec((_BLK, _D), lambda i: (i, 0)),
            pl.BlockSpec((_BLK, 1), lambda i: (i, 0)),
            pl.BlockSpec((_D, _D), lambda i: (0, 0)),
            pl.BlockSpec((1, _D), lambda i: (0, 0)),
        ],
        out_specs=pl.BlockSpec((_BLK, _D), lambda i: (i, 0)),
        out_shape=jax.ShapeDtypeStruct((_N, _D), jnp.float32),
    )(s_parts, hp, dis, W2, b1)


def _final_out(s_parts, hp, dis, b2):
    def body(s_ref, hp_ref, dis_ref, b_ref, o_ref):
        o_ref[...] = dis_ref[...] * (s_ref[0] + s_ref[1] + hp_ref[...]) + b_ref[...]

    return pl.pallas_call(
        body,
        grid=(_NBLK,),
        in_specs=[
            pl.BlockSpec((_NC, _BLK, _D), lambda i: (0, i, 0)),
            pl.BlockSpec((_BLK, _D), lambda i: (i, 0)),
            pl.BlockSpec((_BLK, 1), lambda i: (i, 0)),
            pl.BlockSpec((1, _D), lambda i: (0, 0)),
        ],
        out_specs=pl.BlockSpec((_BLK, _D), lambda i: (i, 0)),
        out_shape=jax.ShapeDtypeStruct((_N, _D), jnp.float32),
    )(s_parts, hp, dis, b2)


def kernel(x, edge_index, edge_weight, W1, b1, W2, b2):
    src = edge_index[0].astype(jnp.int32)
    dst = edge_index[1].astype(jnp.int32)
    ew = edge_weight.astype(jnp.float32)

    src = jnp.pad(src, (0, _E_PAD - _E)).reshape(_NCHUNKS, _CHUNK)
    dst = jnp.pad(dst, (0, _E_PAD - _E)).reshape(_NCHUNKS, _CHUNK)
    ew = jnp.pad(ew, (0, _E_PAD - _E)).reshape(_NCHUNKS, _CHUNK)

    deg_parts = _deg_partials(dst, ew)
    dis = _dis_from_deg(deg_parts).reshape(_N_PAD, 1)[:_N]

    h1p = _mm_scale(x, W1, dis)
    s1 = _scatter_partials(h1p, src, dst, ew)
    h2p = _layer2_mm(s1, h1p, dis, W2, b1.reshape(1, _D))
    s2 = _scatter_partials(h2p, src, dst, ew)
    return _final_out(s2, h2p, dis, b2.reshape(1, _D))


# speedup vs baseline: 1.1102x; 1.1102x over previous
import dataclasses
import functools

import jax
import jax.numpy as jnp
from jax import lax
from jax.experimental import pallas as pl
from jax.experimental.pallas import tpu as pltpu
from jax.experimental.pallas import tpu_sc as plsc

_N = 10000
_E = 320000
_D = 128
_NC = 2
_NS = 16
_NW = _NC * _NS
_N_PAD = 10240
_E_PAD = 327680
_CHUNK = 128
_NCHUNKS = _E_PAD // _CHUNK
_CPT0 = 116
_CPT1 = (_NCHUNKS - _NS * _CPT0) // _NS
_RPT = _N_PAD // _NS
_ARA = 624
_ARB = _N - 15 * _ARA

_BLK = 2000
_NBLK = _N // _BLK


def _sc_mesh():
    return plsc.VectorSubcoreMesh(core_axis_name="c", subcore_axis_name="s")


def _sc_compiler_params():
    cp = pltpu.CompilerParams()
    if "needs_layout_passes" in pltpu.CompilerParams.__dataclass_fields__:
        cp = dataclasses.replace(cp, needs_layout_passes=False)
    return cp


def _deg_partials(dst_r, ew_r):
    grp = 16
    cpt = _NCHUNKS // _NW

    @functools.partial(
        pl.kernel,
        mesh=_sc_mesh(),
        out_type=jax.ShapeDtypeStruct((_NC, _N_PAD), jnp.float32),
        scratch_types=[
            pltpu.VMEM_SHARED((_N_PAD,), jnp.float32),
            pltpu.VMEM((cpt, _CHUNK), jnp.int32),
            pltpu.VMEM((cpt, _CHUNK), jnp.float32),
            pltpu.VMEM((_RPT,), jnp.float32),
            pltpu.SemaphoreType.DMA,
        ],
    )
    def k(dst_hbm, ew_hbm, out_hbm, acc, idx_all, ew_all, zbuf, sem):
        c = lax.axis_index("c")
        s = lax.axis_index("s")
        wid = c * _NS + s

        pltpu.sync_copy(dst_hbm.at[pl.ds(wid * cpt, cpt)], idx_all)
        pltpu.sync_copy(ew_hbm.at[pl.ds(wid * cpt, cpt)], ew_all)

        @pl.loop(0, _RPT // 16)
        def _(i):
            zbuf[pl.ds(i * 16, 16)] = jnp.zeros((16,), jnp.float32)

        pltpu.sync_copy(zbuf, acc.at[pl.ds(s * _RPT, _RPT)])
        plsc.subcore_barrier()

        @pl.loop(0, cpt // grp)
        def _(gi):
            for j in range(grp):
                pltpu.async_copy(ew_all.at[gi * grp + j],
                                 acc.at[idx_all.at[gi * grp + j]], sem,
                                 add=True)
            for j in range(grp):
                pltpu.make_async_copy(ew_all.at[gi * grp + j],
                                      acc.at[idx_all.at[gi * grp + j]],
                                      sem).wait()

        plsc.subcore_barrier()
        pltpu.sync_copy(acc.at[pl.ds(s * _RPT, _RPT)],
                        out_hbm.at[c, pl.ds(s * _RPT, _RPT)])

    return k(dst_r, ew_r)


def _scatter_partials(hp, ed_r):
    @functools.partial(
        pl.kernel,
        mesh=_sc_mesh(),
        out_type=jax.ShapeDtypeStruct((_NC, _N, _D), jnp.float32),
        compiler_params=_sc_compiler_params(),
        scratch_types=[
            pltpu.VMEM_SHARED((_N, _D), jnp.float32),
            pltpu.VMEM((3, _CHUNK), jnp.int32),
            pltpu.VMEM((3, _CHUNK), jnp.int32),
            pltpu.VMEM((3, _CHUNK), jnp.int32),
            pltpu.VMEM((_CHUNK, _D), jnp.float32),
            pltpu.VMEM((_CHUNK, _D), jnp.float32),
            pltpu.VMEM((_CHUNK, _D), jnp.float32),
            pltpu.SemaphoreType.DMA,
            pltpu.SemaphoreType.DMA,
            pltpu.SemaphoreType.DMA,
            pltpu.SemaphoreType.DMA,
            pltpu.SemaphoreType.DMA,
            pltpu.SemaphoreType.DMA,
        ],
    )
    def k(hp_hbm, ed_hbm, out_hbm,
          acc, e0, e1, e2, b0, b1, b2, gs0, gs1, gs2, ss0, ss1, ss2):
        c = lax.axis_index("c")
        s = lax.axis_index("s")
        bufs = (b0, b1, b2)
        edat = (e0, e1, e2)
        gs = (gs0, gs1, gs2)
        ss = (ss0, ss1, ss2)

        @pl.loop(0, _CHUNK)
        def _(i):
            for f in range(_D // 16):
                b0[i, pl.ds(f * 16, 16)] = jnp.zeros((16,), jnp.float32)

        @pl.when(s < 15)
        def _():
            @pl.loop(0, _ARA // 104)
            def _(kk):
                pltpu.sync_copy(b0.at[pl.ds(0, 104)],
                                acc.at[pl.ds(s * _ARA + kk * 104, 104)])

        @pl.when(s == 15)
        def _():
            @pl.loop(0, _ARB // _CHUNK)
            def _(kk):
                pltpu.sync_copy(
                    b0, acc.at[pl.ds(15 * _ARA + kk * _CHUNK, _CHUNK)])

        plsc.subcore_barrier()

        def fetch(ch, j):
            pltpu.async_copy(ed_hbm.at[ch], edat[j], gs[j])

        def wait_fetch(ch, j):
            pltpu.make_async_copy(ed_hbm.at[ch], edat[j], gs[j]).wait()

        def gather_rows(j):
            pltpu.async_copy(hp_hbm.at[edat[j].at[0]], bufs[j], gs[j])

        def wait_rows(j):
            pltpu.make_async_copy(hp_hbm.at[edat[j].at[0]], bufs[j],
                                  gs[j]).wait()

        def scale(j):
            buf = bufs[j]

            @pl.loop(0, _CHUNK)
            def _(e):
                w16i = plsc.load_gather(
                    edat[j], [jnp.full((16,), 2, jnp.int32),
                              jnp.full((16,), e, jnp.int32)])
                w16 = plsc.bitcast(w16i, jnp.float32)
                for f in range(_D // 16):
                    sl = pl.ds(f * 16, 16)
                    buf[e, sl] = buf[e, sl] * w16

        def scatter(j):
            pltpu.async_copy(bufs[j], acc.at[edat[j].at[1]], ss[j], add=True)

        def wait_scatter(j):
            pltpu.make_async_copy(bufs[j], acc.at[edat[j].at[1]],
                                  ss[j]).wait()

        def pipeline(base, cpt):
            fetch(base, 0)
            wait_fetch(base, 0)
            gather_rows(0)
            fetch(base + 1, 1)
            wait_fetch(base + 1, 1)
            gather_rows(1)
            wait_rows(0)
            scale(0)
            fetch(base + 2, 2)
            scatter(0)
            wait_fetch(base + 2, 2)
            gather_rows(2)
            wait_rows(1)
            scale(1)
            wait_scatter(0)
            fetch(base + 3, 0)
            scatter(1)

            @pl.loop(0, (cpt - 2) // 3)
            def _(i):
                for u in range(3):
                    g = 2 + 3 * i + u
                    j = (2 + u) % 3
                    j1 = (j + 1) % 3
                    j2 = (j + 2) % 3
                    nxt = base + jnp.minimum(g + 1, cpt - 1)
                    nx2 = base + jnp.minimum(g + 2, cpt - 1)
                    wait_fetch(nxt, j1)
                    gather_rows(j1)
                    wait_rows(j)
                    scale(j)
                    wait_scatter(j2)
                    fetch(nx2, j2)
                    scatter(j)

            wait_fetch(base + cpt - 1, (cpt + 1) % 3)
            wait_rows(cpt % 3)
            wait_scatter((cpt - 1) % 3)

        @pl.when(c == 0)
        def _():
            pipeline(s * _CPT0, _CPT0)

        @pl.when(c == 1)
        def _():
            pipeline(_NS * _CPT0 + s * _CPT1, _CPT1)

        plsc.subcore_barrier()

        @pl.when(s < 15)
        def _():
            @pl.loop(0, _ARA // 104)
            def _(kk):
                r0 = s * _ARA + kk * 104
                pltpu.sync_copy(acc.at[pl.ds(r0, 104)],
                                out_hbm.at[c, pl.ds(r0, 104)])

        @pl.when(s == 15)
        def _():
            @pl.loop(0, _ARB // _CHUNK)
            def _(kk):
                r0 = 15 * _ARA + kk * _CHUNK
                pltpu.sync_copy(acc.at[pl.ds(r0, _CHUNK)],
                                out_hbm.at[c, pl.ds(r0, _CHUNK)])

    return k(hp, ed_r)


def _dis_from_deg(deg_parts):
    deg_r = deg_parts.reshape(_NC, _N_PAD // 128, 128)

    def body(deg_ref, out_ref):
        out_ref[...] = lax.rsqrt(deg_ref[0] + deg_ref[1] + 1.0)

    return pl.pallas_call(
        body,
        out_shape=jax.ShapeDtypeStruct((_N_PAD // 128, 128), jnp.float32),
    )(deg_r)


def _mm_scale(x, W, dis):
    def body(x_ref, w_ref, dis_ref, o_ref):
        h = jnp.dot(x_ref[...], w_ref[...], preferred_element_type=jnp.float32)
        o_ref[...] = dis_ref[...] * h

    return pl.pallas_call(
        body,
        grid=(_NBLK,),
        in_specs=[
            pl.BlockSpec((_BLK, _D), lambda i: (i, 0)),
            pl.BlockSpec((_D, _D), lambda i: (0, 0)),
            pl.BlockSpec((_BLK, 1), lambda i: (i, 0)),
        ],
        out_specs=pl.BlockSpec((_BLK, _D), lambda i: (i, 0)),
        out_shape=jax.ShapeDtypeStruct((_N, _D), jnp.float32),
    )(x, W, dis)


def _layer2_mm(s_parts, hp, dis, W2, b1):
    def body(s_ref, hp_ref, dis_ref, w_ref, b_ref, o_ref):
        g = dis_ref[...] * (s_ref[0] + s_ref[1] + hp_ref[...]) + b_ref[...]
        g = jnp.maximum(g, 0.0)
        h2 = jnp.dot(g, w_ref[...], preferred_element_type=jnp.float32)
        o_ref[...] = dis_ref[...] * h2

    return pl.pallas_call(
        body,
        grid=(_NBLK,),
        in_specs=[
            pl.BlockSpec((_NC, _BLK, _D), lambda i: (0, i, 0)),
            pl.BlockSpec((_BLK, _D), lambda i: (i, 0)),
            pl.BlockSpec((_BLK, 1), lambda i: (i, 0)),
            pl.BlockSpec((_D, _D), lambda i: (0, 0)),
            pl.BlockSpec((1, _D), lambda i: (0, 0)),
        ],
        out_specs=pl.BlockSpec((_BLK, _D), lambda i: (i, 0)),
        out_shape=jax.ShapeDtypeStruct((_N, _D), jnp.float32),
    )(s_parts, hp, dis, W2, b1)


def _final_out(s_parts, hp, dis, b2):
    def body(s_ref, hp_ref, dis_ref, b_ref, o_ref):
        o_ref[...] = dis_ref[...] * (s_ref[0] + s_ref[1] + hp_ref[...]) + b_ref[...]

    return pl.pallas_call(
        body,
        grid=(_NBLK,),
        in_specs=[
            pl.BlockSpec((_NC, _BLK, _D), lambda i: (0, i, 0)),
            pl.BlockSpec((_BLK, _D), lambda i: (i, 0)),
            pl.BlockSpec((_BLK, 1), lambda i: (i, 0)),
            pl.BlockSpec((1, _D), lambda i: (0, 0)),
        ],
        out_specs=pl.BlockSpec((_BLK, _D), lambda i: (i, 0)),
        out_shape=jax.ShapeDtypeStruct((_N, _D), jnp.float32),
    )(s_parts, hp, dis, b2)


def kernel(x, edge_index, edge_weight, W1, b1, W2, b2):
    src = edge_index[0].astype(jnp.int32)
    dst = edge_index[1].astype(jnp.int32)
    ew = edge_weight.astype(jnp.float32)

    src = jnp.pad(src, (0, _E_PAD - _E)).reshape(_NCHUNKS, _CHUNK)
    dst = jnp.pad(dst, (0, _E_PAD - _E)).reshape(_NCHUNKS, _CHUNK)
    ew = jnp.pad(ew, (0, _E_PAD - _E)).reshape(_NCHUNKS, _CHUNK)
    ewi = jax.lax.bitcast_convert_type(ew, jnp.int32)
    ed = jnp.stack([src, dst, ewi], axis=1)

    deg_parts = _deg_partials(dst, ew)
    dis = _dis_from_deg(deg_parts).reshape(_N_PAD, 1)[:_N]

    h1p = _mm_scale(x, W1, dis)
    s1 = _scatter_partials(h1p, ed)
    h2p = _layer2_mm(s1, h1p, dis, W2, b1.reshape(1, _D))
    s2 = _scatter_partials(h2p, ed)
    return _final_out(s2, h2p, dis, b2.reshape(1, _D))


# speedup vs baseline: 1.1141x; 1.0035x over previous
import dataclasses
import functools

import jax
import jax.numpy as jnp
from jax import lax
from jax.experimental import pallas as pl
from jax.experimental.pallas import tpu as pltpu
from jax.experimental.pallas import tpu_sc as plsc

_N = 10000
_E = 320000
_D = 128
_NC = 2
_NS = 16
_NW = _NC * _NS
_N_PAD = 10240
_E_PAD = 327680
_CHUNK = 128
_NCHUNKS = _E_PAD // _CHUNK
_CPT0 = 134
_CPT1 = (_NCHUNKS - _NS * _CPT0) // _NS
_RPT = _N_PAD // _NS
_ARA = 624
_ARB = _N - 15 * _ARA

_BLK = 2000
_NBLK = _N // _BLK


def _sc_mesh():
    return plsc.VectorSubcoreMesh(core_axis_name="c", subcore_axis_name="s")


def _sc_compiler_params():
    cp = pltpu.CompilerParams()
    if "needs_layout_passes" in pltpu.CompilerParams.__dataclass_fields__:
        cp = dataclasses.replace(cp, needs_layout_passes=False)
    return cp


def _deg_partials(dst_r, ew_r):
    grp = 16
    cpt = _NCHUNKS // _NW

    @functools.partial(
        pl.kernel,
        mesh=_sc_mesh(),
        out_type=jax.ShapeDtypeStruct((_NC, _N_PAD), jnp.float32),
        scratch_types=[
            pltpu.VMEM_SHARED((_N_PAD,), jnp.float32),
            pltpu.VMEM((cpt, _CHUNK), jnp.int32),
            pltpu.VMEM((cpt, _CHUNK), jnp.float32),
            pltpu.VMEM((_RPT,), jnp.float32),
            pltpu.SemaphoreType.DMA,
        ],
    )
    def k(dst_hbm, ew_hbm, out_hbm, acc, idx_all, ew_all, zbuf, sem):
        c = lax.axis_index("c")
        s = lax.axis_index("s")
        wid = c * _NS + s

        pltpu.sync_copy(dst_hbm.at[pl.ds(wid * cpt, cpt)], idx_all)
        pltpu.sync_copy(ew_hbm.at[pl.ds(wid * cpt, cpt)], ew_all)

        @pl.loop(0, _RPT // 16)
        def _(i):
            zbuf[pl.ds(i * 16, 16)] = jnp.zeros((16,), jnp.float32)

        pltpu.sync_copy(zbuf, acc.at[pl.ds(s * _RPT, _RPT)])
        plsc.subcore_barrier()

        @pl.loop(0, cpt // grp)
        def _(gi):
            for j in range(grp):
                pltpu.async_copy(ew_all.at[gi * grp + j],
                                 acc.at[idx_all.at[gi * grp + j]], sem,
                                 add=True)
            for j in range(grp):
                pltpu.make_async_copy(ew_all.at[gi * grp + j],
                                      acc.at[idx_all.at[gi * grp + j]],
                                      sem).wait()

        plsc.subcore_barrier()
        pltpu.sync_copy(acc.at[pl.ds(s * _RPT, _RPT)],
                        out_hbm.at[c, pl.ds(s * _RPT, _RPT)])

    return k(dst_r, ew_r)


def _scatter_partials(hp, ed_r):
    @functools.partial(
        pl.kernel,
        mesh=_sc_mesh(),
        out_type=jax.ShapeDtypeStruct((_NC, _N, _D), jnp.float32),
        compiler_params=_sc_compiler_params(),
        scratch_types=[
            pltpu.VMEM_SHARED((_N, _D), jnp.float32),
            pltpu.VMEM((3, _CHUNK), jnp.int32),
            pltpu.VMEM((3, _CHUNK), jnp.int32),
            pltpu.VMEM((3, _CHUNK), jnp.int32),
            pltpu.VMEM((_CHUNK, _D), jnp.float32),
            pltpu.VMEM((_CHUNK, _D), jnp.float32),
            pltpu.VMEM((_CHUNK, _D), jnp.float32),
            pltpu.SemaphoreType.DMA,
            pltpu.SemaphoreType.DMA,
            pltpu.SemaphoreType.DMA,
            pltpu.SemaphoreType.DMA,
            pltpu.SemaphoreType.DMA,
            pltpu.SemaphoreType.DMA,
        ],
    )
    def k(hp_hbm, ed_hbm, out_hbm,
          acc, e0, e1, e2, b0, b1, b2, gs0, gs1, gs2, ss0, ss1, ss2):
        c = lax.axis_index("c")
        s = lax.axis_index("s")
        bufs = (b0, b1, b2)
        edat = (e0, e1, e2)
        gs = (gs0, gs1, gs2)
        ss = (ss0, ss1, ss2)

        @pl.loop(0, _CHUNK)
        def _(i):
            for f in range(_D // 16):
                b0[i, pl.ds(f * 16, 16)] = jnp.zeros((16,), jnp.float32)

        @pl.when(s < 15)
        def _():
            @pl.loop(0, _ARA // 104)
            def _(kk):
                pltpu.sync_copy(b0.at[pl.ds(0, 104)],
                                acc.at[pl.ds(s * _ARA + kk * 104, 104)])

        @pl.when(s == 15)
        def _():
            @pl.loop(0, _ARB // _CHUNK)
            def _(kk):
                pltpu.sync_copy(
                    b0, acc.at[pl.ds(15 * _ARA + kk * _CHUNK, _CHUNK)])

        plsc.subcore_barrier()

        def fetch(ch, j):
            pltpu.async_copy(ed_hbm.at[ch], edat[j], gs[j])

        def wait_fetch(ch, j):
            pltpu.make_async_copy(ed_hbm.at[ch], edat[j], gs[j]).wait()

        def gather_rows(j):
            pltpu.async_copy(hp_hbm.at[edat[j].at[0]], bufs[j], gs[j])

        def wait_rows(j):
            pltpu.make_async_copy(hp_hbm.at[edat[j].at[0]], bufs[j],
                                  gs[j]).wait()

        def scale(j):
            buf = bufs[j]

            @pl.loop(0, _CHUNK)
            def _(e):
                w16i = plsc.load_gather(
                    edat[j], [jnp.full((16,), 2, jnp.int32),
                              jnp.full((16,), e, jnp.int32)])
                w16 = plsc.bitcast(w16i, jnp.float32)
                for f in range(_D // 16):
                    sl = pl.ds(f * 16, 16)
                    buf[e, sl] = buf[e, sl] * w16

        def scatter(j):
            pltpu.async_copy(bufs[j], acc.at[edat[j].at[1]], ss[j], add=True)

        def wait_scatter(j):
            pltpu.make_async_copy(bufs[j], acc.at[edat[j].at[1]],
                                  ss[j]).wait()

        def pipeline(base, cpt):
            fetch(base, 0)
            wait_fetch(base, 0)
            gather_rows(0)
            fetch(base + 1, 1)
            wait_fetch(base + 1, 1)
            gather_rows(1)
            wait_rows(0)
            scale(0)
            fetch(base + 2, 2)
            scatter(0)
            wait_fetch(base + 2, 2)
            gather_rows(2)
            wait_rows(1)
            scale(1)
            wait_scatter(0)
            fetch(base + 3, 0)
            scatter(1)

            @pl.loop(0, (cpt - 2) // 3)
            def _(i):
                for u in range(3):
                    g = 2 + 3 * i + u
                    j = (2 + u) % 3
                    j1 = (j + 1) % 3
                    j2 = (j + 2) % 3
                    nxt = base + jnp.minimum(g + 1, cpt - 1)
                    nx2 = base + jnp.minimum(g + 2, cpt - 1)
                    wait_fetch(nxt, j1)
                    gather_rows(j1)
                    wait_rows(j)
                    scale(j)
                    wait_scatter(j2)
                    fetch(nx2, j2)
                    scatter(j)

            wait_fetch(base + cpt - 1, (cpt + 1) % 3)
            wait_rows(cpt % 3)
            wait_scatter((cpt - 1) % 3)

        @pl.when(c == 0)
        def _():
            pipeline(s * _CPT0, _CPT0)

        @pl.when(c == 1)
        def _():
            pipeline(_NS * _CPT0 + s * _CPT1, _CPT1)

        plsc.subcore_barrier()

        @pl.when(s < 15)
        def _():
            @pl.loop(0, _ARA // 104)
            def _(kk):
                r0 = s * _ARA + kk * 104
                pltpu.sync_copy(acc.at[pl.ds(r0, 104)],
                                out_hbm.at[c, pl.ds(r0, 104)])

        @pl.when(s == 15)
        def _():
            @pl.loop(0, _ARB // _CHUNK)
            def _(kk):
                r0 = 15 * _ARA + kk * _CHUNK
                pltpu.sync_copy(acc.at[pl.ds(r0, _CHUNK)],
                                out_hbm.at[c, pl.ds(r0, _CHUNK)])

    return k(hp, ed_r)


def _dis_from_deg(deg_parts):
    deg_r = deg_parts.reshape(_NC, _N_PAD // 128, 128)

    def body(deg_ref, out_ref):
        out_ref[...] = lax.rsqrt(deg_ref[0] + deg_ref[1] + 1.0)

    return pl.pallas_call(
        body,
        out_shape=jax.ShapeDtypeStruct((_N_PAD // 128, 128), jnp.float32),
    )(deg_r)


def _mm_scale(x, W, dis):
    def body(x_ref, w_ref, dis_ref, o_ref):
        h = jnp.dot(x_ref[...], w_ref[...], preferred_element_type=jnp.float32)
        o_ref[...] = dis_ref[...] * h

    return pl.pallas_call(
        body,
        grid=(_NBLK,),
        in_specs=[
            pl.BlockSpec((_BLK, _D), lambda i: (i, 0)),
            pl.BlockSpec((_D, _D), lambda i: (0, 0)),
            pl.BlockSpec((_BLK, 1), lambda i: (i, 0)),
        ],
        out_specs=pl.BlockSpec((_BLK, _D), lambda i: (i, 0)),
        out_shape=jax.ShapeDtypeStruct((_N, _D), jnp.float32),
    )(x, W, dis)


def _layer2_mm(s_parts, hp, dis, W2, b1):
    def body(s_ref, hp_ref, dis_ref, w_ref, b_ref, o_ref):
        g = dis_ref[...] * (s_ref[0] + s_ref[1] + hp_ref[...]) + b_ref[...]
        g = jnp.maximum(g, 0.0)
        h2 = jnp.dot(g, w_ref[...], preferred_element_type=jnp.float32)
        o_ref[...] = dis_ref[...] * h2

    return pl.pallas_call(
        body,
        grid=(_NBLK,),
        in_specs=[
            pl.BlockSpec((_NC, _BLK, _D), lambda i: (0, i, 0)),
            pl.BlockSpec((_BLK, _D), lambda i: (i, 0)),
            pl.BlockSpec((_BLK, 1), lambda i: (i, 0)),
            pl.BlockSpec((_D, _D), lambda i: (0, 0)),
            pl.BlockSpec((1, _D), lambda i: (0, 0)),
        ],
        out_specs=pl.BlockSpec((_BLK, _D), lambda i: (i, 0)),
        out_shape=jax.ShapeDtypeStruct((_N, _D), jnp.float32),
    )(s_parts, hp, dis, W2, b1)


def _final_out(s_parts, hp, dis, b2):
    def body(s_ref, hp_ref, dis_ref, b_ref, o_ref):
        o_ref[...] = dis_ref[...] * (s_ref[0] + s_ref[1] + hp_ref[...]) + b_ref[...]

    return pl.pallas_call(
        body,
        grid=(_NBLK,),
        in_specs=[
            pl.BlockSpec((_NC, _BLK, _D), lambda i: (0, i, 0)),
            pl.BlockSpec((_BLK, _D), lambda i: (i, 0)),
            pl.BlockSpec((_BLK, 1), lambda i: (i, 0)),
            pl.BlockSpec((1, _D), lambda i: (0, 0)),
        ],
        out_specs=pl.BlockSpec((_BLK, _D), lambda i: (i, 0)),
        out_shape=jax.ShapeDtypeStruct((_N, _D), jnp.float32),
    )(s_parts, hp, dis, b2)


def kernel(x, edge_index, edge_weight, W1, b1, W2, b2):
    src = edge_index[0].astype(jnp.int32)
    dst = edge_index[1].astype(jnp.int32)
    ew = edge_weight.astype(jnp.float32)

    src = jnp.pad(src, (0, _E_PAD - _E)).reshape(_NCHUNKS, _CHUNK)
    dst = jnp.pad(dst, (0, _E_PAD - _E)).reshape(_NCHUNKS, _CHUNK)
    ew = jnp.pad(ew, (0, _E_PAD - _E)).reshape(_NCHUNKS, _CHUNK)
    ewi = jax.lax.bitcast_convert_type(ew, jnp.int32)
    ed = jnp.stack([src, dst, ewi], axis=1)

    deg_parts = _deg_partials(dst, ew)
    dis = _dis_from_deg(deg_parts).reshape(_N_PAD, 1)[:_N]

    h1p = _mm_scale(x, W1, dis)
    s1 = _scatter_partials(h1p, ed)
    h2p = _layer2_mm(s1, h1p, dis, W2, b1.reshape(1, _D))
    s2 = _scatter_partials(h2p, ed)
    return _final_out(s2, h2p, dis, b2.reshape(1, _D))
